# baseline (speedup 1.0000x reference)
import functools

import jax
import jax.numpy as jnp
from jax import lax
from jax.experimental import pallas as pl
from jax.experimental.pallas import tpu as pltpu
from jax.experimental.pallas import tpu_sc as plsc

_N = 10000
_E = 320000
_D = 128
_H = 256
_DE = 16

_NC = 2
_NS = 16
_L = 16

_EPAD = 327680
_NROWS = 10240


def _mesh():
    return plsc.VectorSubcoreMesh(
        core_axis_name="c", subcore_axis_name="s",
        num_cores=_NC, num_subcores=_NS)


_DW = 64


def _make_seg_sum(n_passes, with_count):
    rows_per_sub = _NROWS // _NS
    nchunks = _EPAD // 128 // _NS
    nslices = _NC * n_passes

    out_type = [jax.ShapeDtypeStruct((_NROWS, _DW), jnp.float32)
                for _ in range(nslices)]
    if with_count:
        out_type.append(jax.ShapeDtypeStruct((_NROWS,), jnp.float32))

    nbuf = 5
    scratch = [
        pltpu.VMEM((nchunks, 128), jnp.int32),
        pltpu.VMEM((nchunks, 128), jnp.int32),
    ] + [pltpu.VMEM((128, _DW), jnp.float32) for _ in range(nbuf)] + [
        pltpu.VMEM((128,), jnp.float32),
        pltpu.VMEM_SHARED((_NROWS, _DW), jnp.float32),
        pltpu.VMEM_SHARED((_NROWS,), jnp.float32),
    ] + [pltpu.SemaphoreType.DMA for _ in range(2 * nbuf)]

    def body(*refs):
        tables = refs[:nslices]
        srcr, dstr, zrows, zvec = refs[nslices:nslices + 4]
        refs = refs[nslices + 4:]
        outs = refs[:nslices]
        refs = refs[nslices:]
        if with_count:
            cnt_out = refs[0]
            refs = refs[1:]
        src_v, dst_v = refs[0], refs[1]
        rb = refs[2:2 + nbuf]
        ones_v, acc, cacc = refs[2 + nbuf:5 + nbuf]
        gsem = refs[5 + nbuf:5 + 2 * nbuf]
        ssem = refs[5 + 2 * nbuf:5 + 3 * nbuf]
        c = lax.axis_index("c")
        s = lax.axis_index("s")
        rbase = s * rows_per_sub
        rows = pl.ds(rbase, rows_per_sub)

        pltpu.sync_copy(srcr.at[pl.ds(s * nchunks, nchunks)], src_v)
        pltpu.sync_copy(dstr.at[pl.ds(s * nchunks, nchunks)], dst_v)
        for g in range(128 // _L):
            ones_v[pl.ds(g * _L, _L)] = jnp.ones((_L,), jnp.float32)
        if with_count:
            pltpu.sync_copy(zvec.at[rows], cacc.at[rows])

        def accumulate(tbl, do_cnt):
            def gather(j, b):
                pltpu.async_copy(tbl.at[src_v.at[j]], rb[b], gsem[b])

            def wait_gather(j, b):
                pltpu.make_async_copy(tbl.at[src_v.at[j]], rb[b],
                                      gsem[b]).wait()

            def start_scatter(j, b):
                pltpu.async_copy(rb[b], acc.at[dst_v.at[j]], ssem[b],
                                 add=True)
                if do_cnt:
                    pltpu.async_copy(ones_v, cacc.at[dst_v.at[j]], ssem[b],
                                     add=True)

            def wait_scatter(j, b):
                pltpu.make_async_copy(rb[b], acc.at[dst_v.at[j]],
                                      ssem[b]).wait()
                if do_cnt:
                    pltpu.make_async_copy(ones_v, cacc.at[dst_v.at[j]],
                                          ssem[b]).wait()

            for b in range(nbuf - 1):
                gather(b, b)

            def step(t, carry):
                for b in range(nbuf):
                    j = nbuf * t + b
                    wait_gather(j, b)
                    start_scatter(j, b)
                    nxt = (b + nbuf - 1) % nbuf

                    @pl.when(j + nbuf - 1 < nchunks)
                    def _():
                        @pl.when(j >= 1)
                        def _():
                            wait_scatter(j - 1, nxt)

                        gather(j + nbuf - 1, nxt)

                return carry

            lax.fori_loop(0, nchunks // nbuf, step, 0)
            for jj in range(nchunks - nbuf, nchunks):
                wait_scatter(jj, jj % nbuf)

        for p in range(n_passes):
            pltpu.sync_copy(zrows.at[rows], acc.at[rows])
            plsc.subcore_barrier()

            @pl.when(c == 0)
            def _():
                accumulate(tables[p], with_count and p == 0)

            @pl.when(c == 1)
            def _():
                accumulate(tables[n_passes + p], False)

            plsc.subcore_barrier()

            @pl.when(c == 0)
            def _():
                pltpu.sync_copy(acc.at[rows], outs[p].at[rows])

            @pl.when(c == 1)
            def _():
                pltpu.sync_copy(acc.at[rows], outs[n_passes + p].at[rows])

            if p + 1 < n_passes:
                plsc.subcore_barrier()

        if with_count:
            @pl.when(c == 0)
            def _():
                pltpu.sync_copy(cacc.at[rows], cnt_out.at[rows])

    return pl.kernel(
        body, out_type=out_type, mesh=_mesh(), scratch_types=scratch,
        compiler_params=pltpu.CompilerParams(use_tc_tiling_on_sc=False))


def _make_edge_logits():
    chunk = 64
    nbuf = 4
    nrows128 = _EPAD // 128 // (_NC * _NS)
    nchunks = 2 * nrows128
    eper = nchunks * chunk
    ngroups = _H // _L

    hw = _H // 2

    out_type = jax.ShapeDtypeStruct((_EPAD,), jnp.float32)
    scratch = [
        pltpu.VMEM((nrows128, 128), jnp.int32),
        pltpu.VMEM((nrows128, 128), jnp.int32),
        pltpu.VMEM((nbuf, chunk, hw), jnp.int32),
        pltpu.VMEM((nbuf, chunk, hw), jnp.int32),
        pltpu.VMEM((nbuf, chunk, hw), jnp.int32),
        pltpu.VMEM((_H + _L,), jnp.float32),
        pltpu.VMEM((eper,), jnp.float32),
    ] + [pltpu.SemaphoreType.DMA for _ in range(nbuf)]

    def body(A, B, C, srcr, dstr, aux, out, *refs):
        src_v, dst_v, abuf, bbuf, cbuf, aux_v, lbuf = refs[:7]
        sems = refs[7:]
        c = lax.axis_index("c")
        s = lax.axis_index("s")
        w = c * _NS + s
        rowbase = w * nrows128

        pltpu.sync_copy(srcr.at[pl.ds(rowbase, nrows128)], src_v)
        pltpu.sync_copy(dstr.at[pl.ds(rowbase, nrows128)], dst_v)
        pltpu.sync_copy(aux, aux_v)
        w2v = [aux_v[pl.ds(g * _L, _L)] for g in range(ngroups)]
        lane = lax.iota(jnp.int32, _L)
        b2s = jnp.take(aux_v[pl.ds(_H, _L)], lane * 0)

        def idx_views(j, q):
            return (src_v.at[j // 2, pl.ds(q * chunk, chunk)],
                    dst_v.at[j // 2, pl.ds(q * chunk, chunk)])

        def gather3(j, q, b):
            si, di = idx_views(j, q)
            pltpu.async_copy(A.at[si], abuf.at[b], sems[b])
            pltpu.async_copy(B.at[di], bbuf.at[b], sems[b])
            pltpu.async_copy(C.at[pl.ds((rowbase * 2 + j) * chunk, chunk)],
                             cbuf.at[b], sems[b])

        def wait3(j, q, b):
            si, di = idx_views(j, q)
            pltpu.make_async_copy(A.at[si], abuf.at[b], sems[b]).wait()
            pltpu.make_async_copy(B.at[di], bbuf.at[b], sems[b]).wait()
            pltpu.make_async_copy(
                C.at[pl.ds((rowbase * 2 + j) * chunk, chunk)],
                cbuf.at[b], sems[b]).wait()

        def compute(j, b):
            def group16(e16, carry):
                def edge(e_i, res):
                    e = e16 * _L + e_i
                    sacc = jnp.zeros((_L,), jnp.float32)
                    for g in range(ngroups // 2):
                        va = plsc.bitcast(abuf[b, e, pl.ds(g * _L, _L)],
                                          jnp.bfloat16)
                        vb = plsc.bitcast(bbuf[b, e, pl.ds(g * _L, _L)],
                                          jnp.bfloat16)
                        vc = plsc.bitcast(cbuf[b, e, pl.ds(g * _L, _L)],
                                          jnp.bfloat16)
                        v = jnp.maximum(va + vb + vc, 0.0)
                        ve, vo = plsc.unpack(v, format=plsc.PackFormat.INTERLEAVED)
                        sacc = sacc + ve * w2v[2 * g] + vo * w2v[2 * g + 1]
                    for sh in (8, 4, 2, 1):
                        sacc = sacc + jnp.take(sacc, jnp.bitwise_xor(lane, sh))
                    res = jnp.where(lane == e_i, sacc, res)
                    return res

                res = lax.fori_loop(0, _L, edge, jnp.zeros((_L,), jnp.float32))
                lbuf[pl.ds(j * chunk + e16 * _L, _L)] = res + b2s
                return carry

            lax.fori_loop(0, chunk // _L, group16, 0)

        for b in range(nbuf - 1):
            gather3(b, b % 2, b)

        def step(t, carry):
            for b in range(nbuf):
                j = nbuf * t + b
                wait3(j, b % 2, b)

                @pl.when(j + nbuf - 1 < nchunks)
                def _():
                    nj = j + nbuf - 1
                    gather3(nj, (b + nbuf - 1) % 2, (b + nbuf - 1) % nbuf)

                compute(j, b)
            return carry

        lax.fori_loop(0, nchunks // nbuf, step, 0)
        pltpu.sync_copy(lbuf, out.at[pl.ds(w * eper, eper)])

    return pl.kernel(
        body, out_type=out_type, mesh=_mesh(), scratch_types=scratch,
        compiler_params=pltpu.CompilerParams(needs_layout_passes=False))


_RB = 1000


def _pack_bf16_pair(x):
    bi = jax.lax.bitcast_convert_type(x, jnp.int32)
    r = (bi + 0x7FFF + ((bi >> 16) & 1)) >> 16
    lo = r[:, : _H // 2] & 0xFFFF
    hi = r[:, _H // 2:] << 16
    return lo | hi


def _tc_layer1(s0, s1, cnt, x, wl_t, wr_t, b):
    def body(s0_r, s1_r, cnt_r, x_r, wl_r, wr_r, b_r, o_r):
        cn = jnp.maximum(cnt_r[...], 1.0)
        agg = jnp.concatenate([s0_r[...], s1_r[...]], axis=1) / cn
        acc = jnp.dot(agg, wl_r[...], preferred_element_type=jnp.float32)
        acc += jnp.dot(x_r[...], wr_r[...], preferred_element_type=jnp.float32)
        o_r[...] = jnp.maximum(acc + b_r[...], 0.0)

    grid = _N // _RB
    return pl.pallas_call(
        body,
        grid=(grid,),
        in_specs=[
            pl.BlockSpec((_RB, _DW), lambda i: (i, 0)),
            pl.BlockSpec((_RB, _DW), lambda i: (i, 0)),
            pl.BlockSpec((_RB, 1), lambda i: (i, 0)),
            pl.BlockSpec((_RB, _D), lambda i: (i, 0)),
            pl.BlockSpec((_D, _H), lambda i: (0, 0)),
            pl.BlockSpec((_D, _H), lambda i: (0, 0)),
            pl.BlockSpec((1, _H), lambda i: (0, 0)),
        ],
        out_specs=pl.BlockSpec((_RB, _H), lambda i: (i, 0)),
        out_shape=jax.ShapeDtypeStruct((_N, _H), jnp.float32),
    )(s0, s1, cnt, x, wl_t, wr_t, b)


def _tc_layer2_ab(sq, cnt, h, wl2_t, wr_t, b2, w1a_t, w1b_t, b1m):
    def body(s0_r, s1_r, s2_r, s3_r, cnt_r, h_r, wl_r, wr_r, b2_r, w1a_r,
             w1b_r, b1_r, oa_r, ob_r):
        cn = jnp.maximum(cnt_r[...], 1.0)
        agg = jnp.concatenate(
            [s0_r[...], s1_r[...], s2_r[...], s3_r[...]], axis=1) / cn
        emb = jnp.dot(agg, wl_r[...], preferred_element_type=jnp.float32)
        emb += jnp.dot(h_r[...], wr_r[...], preferred_element_type=jnp.float32)
        emb += b2_r[...]
        a_full = jnp.dot(emb, w1a_r[...],
                         preferred_element_type=jnp.float32) + b1_r[...]
        b_full = jnp.dot(emb, w1b_r[...],
                         preferred_element_type=jnp.float32)
        oa_r[...] = _pack_bf16_pair(a_full)
        ob_r[...] = _pack_bf16_pair(b_full)

    grid = _N // _RB
    return pl.pallas_call(
        body,
        grid=(grid,),
        in_specs=[
            pl.BlockSpec((_RB, _DW), lambda i: (i, 0)),
            pl.BlockSpec((_RB, _DW), lambda i: (i, 0)),
            pl.BlockSpec((_RB, _DW), lambda i: (i, 0)),
            pl.BlockSpec((_RB, _DW), lambda i: (i, 0)),
            pl.BlockSpec((_RB, 1), lambda i: (i, 0)),
            pl.BlockSpec((_RB, _H), lambda i: (i, 0)),
            pl.BlockSpec((_H, _H), lambda i: (0, 0)),
            pl.BlockSpec((_H, _H), lambda i: (0, 0)),
            pl.BlockSpec((1, _H), lambda i: (0, 0)),
            pl.BlockSpec((_H, _H), lambda i: (0, 0)),
            pl.BlockSpec((_H, _H), lambda i: (0, 0)),
            pl.BlockSpec((1, _H), lambda i: (0, 0)),
        ],
        out_specs=[
            pl.BlockSpec((_RB, _H // 2), lambda i: (i, 0)),
            pl.BlockSpec((_RB, _H // 2), lambda i: (i, 0)),
        ],
        out_shape=[
            jax.ShapeDtypeStruct((_N, _H // 2), jnp.int32),
            jax.ShapeDtypeStruct((_N, _H // 2), jnp.int32),
        ],
    )(*sq, cnt, h, wl2_t, wr_t, b2, w1a_t, w1b_t, b1m)


def _tc_edge_c(ea, w1c_t):
    def body(ea_r, w_r, o_r):
        cv = jnp.dot(ea_r[...], w_r[...], preferred_element_type=jnp.float32)
        o_r[...] = _pack_bf16_pair(cv)

    eb = 4096
    return pl.pallas_call(
        body,
        grid=(_EPAD // eb,),
        in_specs=[
            pl.BlockSpec((eb, _DE), lambda i: (i, 0)),
            pl.BlockSpec((_DE, _H), lambda i: (0, 0)),
        ],
        out_specs=pl.BlockSpec((eb, _H // 2), lambda i: (i, 0)),
        out_shape=jax.ShapeDtypeStruct((_EPAD, _H // 2), jnp.int32),
    )(ea, w1c_t)


def kernel(x, edge_index, edge_attr, conv1_Wl, conv1_Wr, conv1_b,
           conv2_Wl, conv2_Wr, conv2_b, mlp_W1, mlp_b1, mlp_W2, mlp_b2):
    src = edge_index[0]
    dst = edge_index[1]
    pad = _EPAD - _E
    src_p = jnp.concatenate([src, jnp.zeros((pad,), jnp.int32)])
    dst_seg = jnp.concatenate([dst, jnp.full((pad,), _N, jnp.int32)])
    dst_p = jnp.concatenate([dst, jnp.zeros((pad,), jnp.int32)])
    srcr128 = src_p.reshape(_EPAD // 128, 128)
    dstr128 = dst_seg.reshape(_EPAD // 128, 128)
    dstr128e = dst_p.reshape(_EPAD // 128, 128)

    zrows = jnp.zeros((_NROWS, _DW), jnp.float32)
    zv = jnp.zeros((_NROWS,), jnp.float32)

    s1h0, s1h1, cnt = _make_seg_sum(1, True)(
        x[:, :_DW], x[:, _DW:], srcr128, dstr128, zrows, zv)
    cnt2 = cnt.reshape(_NROWS, 1)
    h = _tc_layer1(s1h0, s1h1, cnt2, x, conv1_Wl.T, conv1_Wr.T,
                   conv1_b.reshape(1, _H))

    sq = _make_seg_sum(2, False)(
        h[:, :_DW], h[:, _DW:2 * _DW], h[:, 2 * _DW:3 * _DW], h[:, 3 * _DW:],
        srcr128, dstr128, zrows, zv)
    w1a_t = mlp_W1[:, :_H].T
    w1b_t = mlp_W1[:, _H:2 * _H].T
    w1c_t = mlp_W1[:, 2 * _H:].T
    a_tab, b_tab = _tc_layer2_ab(
        sq, cnt2, h, conv2_Wl.T, conv2_Wr.T,
        conv2_b.reshape(1, _H), w1a_t, w1b_t, mlp_b1.reshape(1, _H))

    ea_p = jnp.concatenate(
        [edge_attr, jnp.zeros((pad, _DE), jnp.float32)], axis=0)
    c_tab = _tc_edge_c(ea_p, w1c_t)

    w2p = mlp_W2[0].reshape(2, _H // 32, _L).transpose(1, 0, 2).reshape(_H)
    aux = jnp.concatenate([w2p, mlp_b2, jnp.zeros((_L - 1,), jnp.float32)])
    logits = _make_edge_logits()(a_tab, b_tab, c_tab, srcr128, dstr128e, aux)
    return logits[:_E]


# speedup vs baseline: 1.1097x; 1.1097x over previous
import functools

import jax
import jax.numpy as jnp
from jax import lax
from jax.experimental import pallas as pl
from jax.experimental.pallas import tpu as pltpu
from jax.experimental.pallas import tpu_sc as plsc

_N = 10000
_E = 320000
_D = 128
_H = 256
_DE = 16

_NC = 2
_NS = 16
_L = 16

_EPAD = 327680
_NROWS = 10112


def _mesh():
    return plsc.VectorSubcoreMesh(
        core_axis_name="c", subcore_axis_name="s",
        num_cores=_NC, num_subcores=_NS)


_DW = 64


def _make_seg_sum(n_passes, with_count, packed):
    rows_per_sub = _NROWS // _NS
    nchunks = _EPAD // 128 // _NS
    nslices = _NC * n_passes

    out_type = [jax.ShapeDtypeStruct((_NROWS, _DW), jnp.float32)
                for _ in range(nslices)]
    if with_count:
        out_type.append(jax.ShapeDtypeStruct((_NROWS,), jnp.float32))

    nbuf = 4
    scratch = [
        pltpu.VMEM((nchunks, 128), jnp.int32),
        pltpu.VMEM((nchunks, 128), jnp.int32),
    ]
    if packed:
        scratch += [pltpu.VMEM((128, _DW // 2), jnp.int32)
                    for _ in range(nbuf)]
    scratch += [pltpu.VMEM((128, _DW), jnp.float32) for _ in range(nbuf)] + [
        pltpu.VMEM((128,), jnp.float32),
        pltpu.VMEM_SHARED((_NROWS, _DW), jnp.float32),
    ]
    if with_count:
        scratch.append(pltpu.VMEM_SHARED((_NROWS,), jnp.float32))
    scratch += [pltpu.SemaphoreType.DMA for _ in range(2 * nbuf)]

    def body(*refs):
        tables = refs[:nslices]
        srcr, dstr, zrows, zvec = refs[nslices:nslices + 4]
        refs = refs[nslices + 4:]
        outs = refs[:nslices]
        refs = refs[nslices:]
        if with_count:
            cnt_out = refs[0]
            refs = refs[1:]
        src_v, dst_v = refs[0], refs[1]
        refs = refs[2:]
        if packed:
            rp = refs[:nbuf]
            refs = refs[nbuf:]
        rb = refs[:nbuf]
        refs = refs[nbuf:]
        ones_v, acc = refs[0], refs[1]
        refs = refs[2:]
        if with_count:
            cacc = refs[0]
            refs = refs[1:]
        gsem = refs[:nbuf]
        ssem = refs[nbuf:2 * nbuf]
        c = lax.axis_index("c")
        s = lax.axis_index("s")
        rbase = s * rows_per_sub
        rows = pl.ds(rbase, rows_per_sub)

        pltpu.sync_copy(srcr.at[pl.ds(s * nchunks, nchunks)], src_v)
        pltpu.sync_copy(dstr.at[pl.ds(s * nchunks, nchunks)], dst_v)
        for g in range(128 // _L):
            ones_v[pl.ds(g * _L, _L)] = jnp.ones((_L,), jnp.float32)
        if with_count:
            pltpu.sync_copy(zvec.at[rows], cacc.at[rows])

        def accumulate(tbl, do_cnt):
            gdst = rp if packed else rb

            def gather(j, b):
                pltpu.async_copy(tbl.at[src_v.at[j]], gdst[b], gsem[b])

            def wait_gather(j, b):
                pltpu.make_async_copy(tbl.at[src_v.at[j]], gdst[b],
                                      gsem[b]).wait()

            def expand(b):
                def row(r, carry):
                    for g in range(_DW // 32):
                        v = plsc.bitcast(rp[b][r, pl.ds(g * _L, _L)],
                                         jnp.bfloat16)
                        ve, vo = plsc.unpack(
                            v, format=plsc.PackFormat.INTERLEAVED)
                        rb[b][r, pl.ds(g * _L, _L)] = ve
                        rb[b][r, pl.ds(_DW // 2 + g * _L, _L)] = vo
                    return carry

                lax.fori_loop(0, 128, row, 0)

            def start_scatter(j, b):
                pltpu.async_copy(rb[b], acc.at[dst_v.at[j]], ssem[b],
                                 add=True)
                if do_cnt:
                    pltpu.async_copy(ones_v, cacc.at[dst_v.at[j]], ssem[b],
                                     add=True)

            def wait_scatter(j, b):
                pltpu.make_async_copy(rb[b], acc.at[dst_v.at[j]],
                                      ssem[b]).wait()
                if do_cnt:
                    pltpu.make_async_copy(ones_v, cacc.at[dst_v.at[j]],
                                          ssem[b]).wait()

            for b in range(nbuf - 1):
                gather(b, b)

            def step(t, carry):
                for b in range(nbuf):
                    j = nbuf * t + b
                    wait_gather(j, b)
                    if packed:
                        expand(b)
                    start_scatter(j, b)
                    nxt = (b + nbuf - 1) % nbuf

                    @pl.when(j + nbuf - 1 < nchunks)
                    def _():
                        @pl.when(j >= 1)
                        def _():
                            wait_scatter(j - 1, nxt)

                        gather(j + nbuf - 1, nxt)

                return carry

            lax.fori_loop(0, nchunks // nbuf, step, 0)
            for jj in range(nchunks - nbuf, nchunks):
                wait_scatter(jj, jj % nbuf)

        for p in range(n_passes):
            pltpu.sync_copy(zrows.at[rows], acc.at[rows])
            plsc.subcore_barrier()

            @pl.when(c == 0)
            def _():
                accumulate(tables[p], with_count and p == 0)

            @pl.when(c == 1)
            def _():
                accumulate(tables[n_passes + p], False)

            plsc.subcore_barrier()

            @pl.when(c == 0)
            def _():
                pltpu.sync_copy(acc.at[rows], outs[p].at[rows])

            @pl.when(c == 1)
            def _():
                pltpu.sync_copy(acc.at[rows], outs[n_passes + p].at[rows])

            if p + 1 < n_passes:
                plsc.subcore_barrier()

        if with_count:
            @pl.when(c == 0)
            def _():
                pltpu.sync_copy(cacc.at[rows], cnt_out.at[rows])

    return pl.kernel(
        body, out_type=out_type, mesh=_mesh(), scratch_types=scratch,
        compiler_params=pltpu.CompilerParams(
            use_tc_tiling_on_sc=False, needs_layout_passes=not packed))


def _make_edge_logits():
    chunk = 64
    nbuf = 4
    nrows128 = _EPAD // 128 // (_NC * _NS)
    nchunks = 2 * nrows128
    eper = nchunks * chunk
    ngroups = _H // _L

    hw = _H // 2

    out_type = jax.ShapeDtypeStruct((_EPAD,), jnp.float32)
    scratch = [
        pltpu.VMEM((nrows128, 128), jnp.int32),
        pltpu.VMEM((nrows128, 128), jnp.int32),
        pltpu.VMEM((nbuf, chunk, hw), jnp.int32),
        pltpu.VMEM((nbuf, chunk, hw), jnp.int32),
        pltpu.VMEM((nbuf, chunk, hw), jnp.int32),
        pltpu.VMEM((_H + _L,), jnp.float32),
        pltpu.VMEM((eper,), jnp.float32),
    ] + [pltpu.SemaphoreType.DMA for _ in range(nbuf)]

    def body(A, B, C, srcr, dstr, aux, out, *refs):
        src_v, dst_v, abuf, bbuf, cbuf, aux_v, lbuf = refs[:7]
        sems = refs[7:]
        c = lax.axis_index("c")
        s = lax.axis_index("s")
        w = c * _NS + s
        rowbase = w * nrows128

        pltpu.sync_copy(srcr.at[pl.ds(rowbase, nrows128)], src_v)
        pltpu.sync_copy(dstr.at[pl.ds(rowbase, nrows128)], dst_v)
        pltpu.sync_copy(aux, aux_v)
        w2v = [aux_v[pl.ds(g * _L, _L)] for g in range(ngroups)]
        lane = lax.iota(jnp.int32, _L)
        b2s = jnp.take(aux_v[pl.ds(_H, _L)], lane * 0)

        def idx_views(j, q):
            return (src_v.at[j // 2, pl.ds(q * chunk, chunk)],
                    dst_v.at[j // 2, pl.ds(q * chunk, chunk)])

        def gather3(j, q, b):
            si, di = idx_views(j, q)
            pltpu.async_copy(A.at[si], abuf.at[b], sems[b])
            pltpu.async_copy(B.at[di], bbuf.at[b], sems[b])
            pltpu.async_copy(C.at[pl.ds((rowbase * 2 + j) * chunk, chunk)],
                             cbuf.at[b], sems[b])

        def wait3(j, q, b):
            si, di = idx_views(j, q)
            pltpu.make_async_copy(A.at[si], abuf.at[b], sems[b]).wait()
            pltpu.make_async_copy(B.at[di], bbuf.at[b], sems[b]).wait()
            pltpu.make_async_copy(
                C.at[pl.ds((rowbase * 2 + j) * chunk, chunk)],
                cbuf.at[b], sems[b]).wait()

        def compute(j, b):
            def group16(e16, carry):
                def edge(e_i, res):
                    e = e16 * _L + e_i
                    sacc = jnp.zeros((_L,), jnp.float32)
                    for g in range(ngroups // 2):
                        va = plsc.bitcast(abuf[b, e, pl.ds(g * _L, _L)],
                                          jnp.bfloat16)
                        vb = plsc.bitcast(bbuf[b, e, pl.ds(g * _L, _L)],
                                          jnp.bfloat16)
                        vc = plsc.bitcast(cbuf[b, e, pl.ds(g * _L, _L)],
                                          jnp.bfloat16)
                        v = jnp.maximum(va + vb + vc, 0.0)
                        ve, vo = plsc.unpack(v, format=plsc.PackFormat.INTERLEAVED)
                        sacc = sacc + ve * w2v[2 * g] + vo * w2v[2 * g + 1]
                    for sh in (8, 4, 2, 1):
                        sacc = sacc + jnp.take(sacc, jnp.bitwise_xor(lane, sh))
                    res = jnp.where(lane == e_i, sacc, res)
                    return res

                res = lax.fori_loop(0, _L, edge, jnp.zeros((_L,), jnp.float32))
                lbuf[pl.ds(j * chunk + e16 * _L, _L)] = res + b2s
                return carry

            lax.fori_loop(0, chunk // _L, group16, 0)

        for b in range(nbuf - 1):
            gather3(b, b % 2, b)

        def step(t, carry):
            for b in range(nbuf):
                j = nbuf * t + b
                wait3(j, b % 2, b)

                @pl.when(j + nbuf - 1 < nchunks)
                def _():
                    nj = j + nbuf - 1
                    gather3(nj, (b + nbuf - 1) % 2, (b + nbuf - 1) % nbuf)

                compute(j, b)
            return carry

        lax.fori_loop(0, nchunks // nbuf, step, 0)
        pltpu.sync_copy(lbuf, out.at[pl.ds(w * eper, eper)])

    return pl.kernel(
        body, out_type=out_type, mesh=_mesh(), scratch_types=scratch,
        compiler_params=pltpu.CompilerParams(needs_layout_passes=False))


_RB = 1000


def _pack_bf16(x, group):
    bi = jax.lax.bitcast_convert_type(x, jnp.int32)
    r = (bi + 0x7FFF + ((bi >> 16) & 1)) >> 16
    hwid = group // 2
    parts = []
    for g in range(0, x.shape[-1], group):
        parts.append((r[:, g:g + hwid] & 0xFFFF)
                     | (r[:, g + hwid:g + group] << 16))
    return parts[0] if len(parts) == 1 else jnp.concatenate(parts, axis=1)


def _pack_bf16_pair(x):
    return _pack_bf16(x, _H)


def _tc_layer1(s0, s1, cnt, x, wl_t, wr_t, b):
    def body(s0_r, s1_r, cnt_r, x_r, wl_r, wr_r, b_r, o_r, op_r):
        cn = jnp.maximum(cnt_r[...], 1.0)
        agg = jnp.concatenate([s0_r[...], s1_r[...]], axis=1) / cn
        acc = jnp.dot(agg, wl_r[...], preferred_element_type=jnp.float32)
        acc += jnp.dot(x_r[...], wr_r[...], preferred_element_type=jnp.float32)
        h = jnp.maximum(acc + b_r[...], 0.0)
        o_r[...] = h
        op_r[...] = _pack_bf16(h, _DW)

    grid = _N // _RB
    return pl.pallas_call(
        body,
        grid=(grid,),
        in_specs=[
            pl.BlockSpec((_RB, _DW), lambda i: (i, 0)),
            pl.BlockSpec((_RB, _DW), lambda i: (i, 0)),
            pl.BlockSpec((_RB, 1), lambda i: (i, 0)),
            pl.BlockSpec((_RB, _D), lambda i: (i, 0)),
            pl.BlockSpec((_D, _H), lambda i: (0, 0)),
            pl.BlockSpec((_D, _H), lambda i: (0, 0)),
            pl.BlockSpec((1, _H), lambda i: (0, 0)),
        ],
        out_specs=[
            pl.BlockSpec((_RB, _H), lambda i: (i, 0)),
            pl.BlockSpec((_RB, _H // 2), lambda i: (i, 0)),
        ],
        out_shape=[
            jax.ShapeDtypeStruct((_N, _H), jnp.float32),
            jax.ShapeDtypeStruct((_N, _H // 2), jnp.int32),
        ],
    )(s0, s1, cnt, x, wl_t, wr_t, b)


def _tc_layer2_ab(sq, cnt, h, wl2_t, wr_t, b2, w1a_t, w1b_t, b1m):
    def body(s0_r, s1_r, s2_r, s3_r, cnt_r, h_r, wl_r, wr_r, b2_r, w1a_r,
             w1b_r, b1_r, oa_r, ob_r):
        cn = jnp.maximum(cnt_r[...], 1.0)
        agg = jnp.concatenate(
            [s0_r[...], s1_r[...], s2_r[...], s3_r[...]], axis=1) / cn
        emb = jnp.dot(agg, wl_r[...], preferred_element_type=jnp.float32)
        emb += jnp.dot(h_r[...], wr_r[...], preferred_element_type=jnp.float32)
        emb += b2_r[...]
        a_full = jnp.dot(emb, w1a_r[...],
                         preferred_element_type=jnp.float32) + b1_r[...]
        b_full = jnp.dot(emb, w1b_r[...],
                         preferred_element_type=jnp.float32)
        oa_r[...] = _pack_bf16_pair(a_full)
        ob_r[...] = _pack_bf16_pair(b_full)

    grid = _N // _RB
    return pl.pallas_call(
        body,
        grid=(grid,),
        in_specs=[
            pl.BlockSpec((_RB, _DW), lambda i: (i, 0)),
            pl.BlockSpec((_RB, _DW), lambda i: (i, 0)),
            pl.BlockSpec((_RB, _DW), lambda i: (i, 0)),
            pl.BlockSpec((_RB, _DW), lambda i: (i, 0)),
            pl.BlockSpec((_RB, 1), lambda i: (i, 0)),
            pl.BlockSpec((_RB, _H), lambda i: (i, 0)),
            pl.BlockSpec((_H, _H), lambda i: (0, 0)),
            pl.BlockSpec((_H, _H), lambda i: (0, 0)),
            pl.BlockSpec((1, _H), lambda i: (0, 0)),
            pl.BlockSpec((_H, _H), lambda i: (0, 0)),
            pl.BlockSpec((_H, _H), lambda i: (0, 0)),
            pl.BlockSpec((1, _H), lambda i: (0, 0)),
        ],
        out_specs=[
            pl.BlockSpec((_RB, _H // 2), lambda i: (i, 0)),
            pl.BlockSpec((_RB, _H // 2), lambda i: (i, 0)),
        ],
        out_shape=[
            jax.ShapeDtypeStruct((_N, _H // 2), jnp.int32),
            jax.ShapeDtypeStruct((_N, _H // 2), jnp.int32),
        ],
    )(*sq, cnt, h, wl2_t, wr_t, b2, w1a_t, w1b_t, b1m)


def _tc_edge_c(ea, w1c_t):
    def body(ea_r, w_r, o_r):
        cv = jnp.dot(ea_r[...], w_r[...], preferred_element_type=jnp.float32)
        o_r[...] = _pack_bf16_pair(cv)

    eb = 4096
    return pl.pallas_call(
        body,
        grid=(_EPAD // eb,),
        in_specs=[
            pl.BlockSpec((eb, _DE), lambda i: (i, 0)),
            pl.BlockSpec((_DE, _H), lambda i: (0, 0)),
        ],
        out_specs=pl.BlockSpec((eb, _H // 2), lambda i: (i, 0)),
        out_shape=jax.ShapeDtypeStruct((_EPAD, _H // 2), jnp.int32),
    )(ea, w1c_t)


def kernel(x, edge_index, edge_attr, conv1_Wl, conv1_Wr, conv1_b,
           conv2_Wl, conv2_Wr, conv2_b, mlp_W1, mlp_b1, mlp_W2, mlp_b2):
    src = edge_index[0]
    dst = edge_index[1]
    pad = _EPAD - _E
    src_p = jnp.concatenate([src, jnp.zeros((pad,), jnp.int32)])
    dst_seg = jnp.concatenate([dst, jnp.full((pad,), _N, jnp.int32)])
    dst_p = jnp.concatenate([dst, jnp.zeros((pad,), jnp.int32)])
    srcr128 = src_p.reshape(_EPAD // 128, 128)
    dstr128 = dst_seg.reshape(_EPAD // 128, 128)
    dstr128e = dst_p.reshape(_EPAD // 128, 128)

    zrows = jnp.zeros((_NROWS, _DW), jnp.float32)
    zv = jnp.zeros((_NROWS,), jnp.float32)

    hw = _DW // 2
    s1h0, s1h1, cnt = _make_seg_sum(1, True, False)(
        x[:, :_DW], x[:, _DW:], srcr128, dstr128, zrows, zv)
    cnt2 = cnt.reshape(_NROWS, 1)
    h, hp = _tc_layer1(s1h0, s1h1, cnt2, x, conv1_Wl.T, conv1_Wr.T,
                       conv1_b.reshape(1, _H))

    sq = _make_seg_sum(2, False, True)(
        hp[:, :hw], hp[:, hw:2 * hw], hp[:, 2 * hw:3 * hw], hp[:, 3 * hw:],
        srcr128, dstr128, zrows, zv)
    w1a_t = mlp_W1[:, :_H].T
    w1b_t = mlp_W1[:, _H:2 * _H].T
    w1c_t = mlp_W1[:, 2 * _H:].T
    a_tab, b_tab = _tc_layer2_ab(
        sq, cnt2, h, conv2_Wl.T, conv2_Wr.T,
        conv2_b.reshape(1, _H), w1a_t, w1b_t, mlp_b1.reshape(1, _H))

    ea_p = jnp.concatenate(
        [edge_attr, jnp.zeros((pad, _DE), jnp.float32)], axis=0)
    c_tab = _tc_edge_c(ea_p, w1c_t)

    w2p = mlp_W2[0].reshape(2, _H // 32, _L).transpose(1, 0, 2).reshape(_H)
    aux = jnp.concatenate([w2p, mlp_b2, jnp.zeros((_L - 1,), jnp.float32)])
    logits = _make_edge_logits()(a_tab, b_tab, c_tab, srcr128, dstr128e, aux)
    return logits[:_E]


# speedup vs baseline: 1.1117x; 1.0018x over previous
import functools

import jax
import jax.numpy as jnp
from jax import lax
from jax.experimental import pallas as pl
from jax.experimental.pallas import tpu as pltpu
from jax.experimental.pallas import tpu_sc as plsc

_N = 10000
_E = 320000
_D = 128
_H = 256
_DE = 16

_NC = 2
_NS = 16
_L = 16

_EPAD = 327680
_NROWS = 10112


def _mesh():
    return plsc.VectorSubcoreMesh(
        core_axis_name="c", subcore_axis_name="s",
        num_cores=_NC, num_subcores=_NS)


_DW = 64


def _make_seg_sum(n_passes, with_count, packed):
    rows_per_sub = _NROWS // _NS
    nchunks = _EPAD // 128 // _NS
    nslices = _NC * n_passes

    out_type = [jax.ShapeDtypeStruct((_NROWS, _DW), jnp.float32)
                for _ in range(nslices)]
    if with_count:
        out_type.append(jax.ShapeDtypeStruct((_NROWS,), jnp.float32))

    nbuf = 4
    scratch = [
        pltpu.VMEM((nchunks, 128), jnp.int32),
        pltpu.VMEM((nchunks, 128), jnp.int32),
    ]
    if packed:
        scratch += [pltpu.VMEM((128, _DW // 2), jnp.int32)
                    for _ in range(nbuf)]
    scratch += [pltpu.VMEM((128, _DW), jnp.float32) for _ in range(nbuf)] + [
        pltpu.VMEM((128,), jnp.float32),
        pltpu.VMEM_SHARED((_NROWS, _DW), jnp.float32),
    ]
    if with_count:
        scratch.append(pltpu.VMEM_SHARED((_NROWS,), jnp.float32))
    scratch += [pltpu.SemaphoreType.DMA for _ in range(2 * nbuf)]

    def body(*refs):
        tables = refs[:nslices]
        srcr, dstr, zrows, zvec = refs[nslices:nslices + 4]
        refs = refs[nslices + 4:]
        outs = refs[:nslices]
        refs = refs[nslices:]
        if with_count:
            cnt_out = refs[0]
            refs = refs[1:]
        src_v, dst_v = refs[0], refs[1]
        refs = refs[2:]
        if packed:
            rp = refs[:nbuf]
            refs = refs[nbuf:]
        rb = refs[:nbuf]
        refs = refs[nbuf:]
        ones_v, acc = refs[0], refs[1]
        refs = refs[2:]
        if with_count:
            cacc = refs[0]
            refs = refs[1:]
        gsem = refs[:nbuf]
        ssem = refs[nbuf:2 * nbuf]
        c = lax.axis_index("c")
        s = lax.axis_index("s")
        rbase = s * rows_per_sub
        rows = pl.ds(rbase, rows_per_sub)

        pltpu.sync_copy(srcr.at[pl.ds(s * nchunks, nchunks)], src_v)
        pltpu.sync_copy(dstr.at[pl.ds(s * nchunks, nchunks)], dst_v)
        for g in range(128 // _L):
            ones_v[pl.ds(g * _L, _L)] = jnp.ones((_L,), jnp.float32)
        if with_count:
            pltpu.sync_copy(zvec.at[rows], cacc.at[rows])

        def accumulate(tbl, do_cnt):
            gdst = rp if packed else rb

            def gather(j, b):
                pltpu.async_copy(tbl.at[src_v.at[j]], gdst[b], gsem[b])

            def wait_gather(j, b):
                pltpu.make_async_copy(tbl.at[src_v.at[j]], gdst[b],
                                      gsem[b]).wait()

            def expand(b):
                def row(r, carry):
                    for g in range(_DW // 32):
                        v = plsc.bitcast(rp[b][r, pl.ds(g * _L, _L)],
                                         jnp.bfloat16)
                        ve, vo = plsc.unpack(
                            v, format=plsc.PackFormat.INTERLEAVED)
                        rb[b][r, pl.ds(g * _L, _L)] = ve
                        rb[b][r, pl.ds(_DW // 2 + g * _L, _L)] = vo
                    return carry

                lax.fori_loop(0, 128, row, 0)

            def start_scatter(j, b):
                pltpu.async_copy(rb[b], acc.at[dst_v.at[j]], ssem[b],
                                 add=True)
                if do_cnt:
                    pltpu.async_copy(ones_v, cacc.at[dst_v.at[j]], ssem[b],
                                     add=True)

            def wait_scatter(j, b):
                pltpu.make_async_copy(rb[b], acc.at[dst_v.at[j]],
                                      ssem[b]).wait()
                if do_cnt:
                    pltpu.make_async_copy(ones_v, cacc.at[dst_v.at[j]],
                                          ssem[b]).wait()

            for b in range(nbuf - 1):
                gather(b, b)

            def step(t, carry):
                for b in range(nbuf):
                    j = nbuf * t + b
                    wait_gather(j, b)
                    if packed:
                        expand(b)
                    start_scatter(j, b)
                    nxt = (b + nbuf - 1) % nbuf

                    @pl.when(j + nbuf - 1 < nchunks)
                    def _():
                        @pl.when(j >= 1)
                        def _():
                            wait_scatter(j - 1, nxt)

                        gather(j + nbuf - 1, nxt)

                return carry

            lax.fori_loop(0, nchunks // nbuf, step, 0)
            for jj in range(nchunks - nbuf, nchunks):
                wait_scatter(jj, jj % nbuf)

        for p in range(n_passes):
            pltpu.sync_copy(zrows.at[rows], acc.at[rows])
            plsc.subcore_barrier()

            @pl.when(c == 0)
            def _():
                accumulate(tables[p], with_count and p == 0)

            @pl.when(c == 1)
            def _():
                accumulate(tables[n_passes + p], False)

            plsc.subcore_barrier()

            @pl.when(c == 0)
            def _():
                pltpu.sync_copy(acc.at[rows], outs[p].at[rows])

            @pl.when(c == 1)
            def _():
                pltpu.sync_copy(acc.at[rows], outs[n_passes + p].at[rows])

            if p + 1 < n_passes:
                plsc.subcore_barrier()

        if with_count:
            @pl.when(c == 0)
            def _():
                pltpu.sync_copy(cacc.at[rows], cnt_out.at[rows])

    return pl.kernel(
        body, out_type=out_type, mesh=_mesh(), scratch_types=scratch,
        compiler_params=pltpu.CompilerParams(
            use_tc_tiling_on_sc=False, needs_layout_passes=not packed))


def _make_edge_logits():
    chunk = 64
    nbuf = 4
    nrows128 = _EPAD // 128 // (_NC * _NS)
    nchunks = 2 * nrows128
    eper = nchunks * chunk
    ngroups = _H // _L

    hw = _H // 2

    out_type = jax.ShapeDtypeStruct((_EPAD,), jnp.float32)
    scratch = [
        pltpu.VMEM((nrows128, 128), jnp.int32),
        pltpu.VMEM((nrows128, 128), jnp.int32),
        pltpu.VMEM((nbuf, chunk, hw), jnp.int32),
        pltpu.VMEM((nbuf, chunk, hw), jnp.int32),
        pltpu.VMEM((nbuf, chunk, hw), jnp.int32),
        pltpu.VMEM((_H + _L,), jnp.float32),
        pltpu.VMEM((eper,), jnp.float32),
    ] + [pltpu.SemaphoreType.DMA for _ in range(nbuf)]

    def body(A, B, C, srcr, dstr, aux, out, *refs):
        src_v, dst_v, abuf, bbuf, cbuf, aux_v, lbuf = refs[:7]
        sems = refs[7:]
        c = lax.axis_index("c")
        s = lax.axis_index("s")
        w = s * _NC + c
        rowbase = w * nrows128

        pltpu.sync_copy(srcr.at[pl.ds(rowbase, nrows128)], src_v)
        pltpu.sync_copy(dstr.at[pl.ds(rowbase, nrows128)], dst_v)
        pltpu.sync_copy(aux, aux_v)
        w2v = [aux_v[pl.ds(g * _L, _L)] for g in range(ngroups)]
        lane = lax.iota(jnp.int32, _L)
        b2s = jnp.take(aux_v[pl.ds(_H, _L)], lane * 0)

        def idx_views(j, q):
            return (src_v.at[j // 2, pl.ds(q * chunk, chunk)],
                    dst_v.at[j // 2, pl.ds(q * chunk, chunk)])

        def gather3(j, q, b):
            si, di = idx_views(j, q)
            pltpu.async_copy(A.at[si], abuf.at[b], sems[b])
            pltpu.async_copy(B.at[di], bbuf.at[b], sems[b])
            pltpu.async_copy(C.at[pl.ds((rowbase * 2 + j) * chunk, chunk)],
                             cbuf.at[b], sems[b])

        def wait3(j, q, b):
            si, di = idx_views(j, q)
            pltpu.make_async_copy(A.at[si], abuf.at[b], sems[b]).wait()
            pltpu.make_async_copy(B.at[di], bbuf.at[b], sems[b]).wait()
            pltpu.make_async_copy(
                C.at[pl.ds((rowbase * 2 + j) * chunk, chunk)],
                cbuf.at[b], sems[b]).wait()

        def compute(j, b):
            def group16(e16, carry):
                def edge(e_i, res):
                    e = e16 * _L + e_i
                    sacc = jnp.zeros((_L,), jnp.float32)
                    for g in range(ngroups // 2):
                        va = plsc.bitcast(abuf[b, e, pl.ds(g * _L, _L)],
                                          jnp.bfloat16)
                        vb = plsc.bitcast(bbuf[b, e, pl.ds(g * _L, _L)],
                                          jnp.bfloat16)
                        vc = plsc.bitcast(cbuf[b, e, pl.ds(g * _L, _L)],
                                          jnp.bfloat16)
                        v = jnp.maximum(va + vb + vc, 0.0)
                        ve, vo = plsc.unpack(v, format=plsc.PackFormat.INTERLEAVED)
                        sacc = sacc + ve * w2v[2 * g] + vo * w2v[2 * g + 1]
                    for sh in (8, 4, 2, 1):
                        sacc = sacc + jnp.take(sacc, jnp.bitwise_xor(lane, sh))
                    res = jnp.where(lane == e_i, sacc, res)
                    return res

                res = lax.fori_loop(0, _L, edge, jnp.zeros((_L,), jnp.float32))
                lbuf[pl.ds(j * chunk + e16 * _L, _L)] = res + b2s
                return carry

            lax.fori_loop(0, chunk // _L, group16, 0)

        for b in range(nbuf - 1):
            gather3(b, b % 2, b)

        def step(t, carry):
            for b in range(nbuf):
                j = nbuf * t + b
                wait3(j, b % 2, b)

                @pl.when(j + nbuf - 1 < nchunks)
                def _():
                    nj = j + nbuf - 1
                    gather3(nj, (b + nbuf - 1) % 2, (b + nbuf - 1) % nbuf)

                compute(j, b)
            return carry

        lax.fori_loop(0, nchunks // nbuf, step, 0)
        pltpu.sync_copy(lbuf, out.at[pl.ds(w * eper, eper)])

    return pl.kernel(
        body, out_type=out_type, mesh=_mesh(), scratch_types=scratch,
        compiler_params=pltpu.CompilerParams(needs_layout_passes=False))


_RB = 1000


def _pack_bf16(x, group):
    bi = jax.lax.bitcast_convert_type(x, jnp.int32)
    r = (bi + 0x7FFF + ((bi >> 16) & 1)) >> 16
    hwid = group // 2
    parts = []
    for g in range(0, x.shape[-1], group):
        parts.append((r[:, g:g + hwid] & 0xFFFF)
                     | (r[:, g + hwid:g + group] << 16))
    return parts[0] if len(parts) == 1 else jnp.concatenate(parts, axis=1)


def _pack_bf16_pair(x):
    return _pack_bf16(x, _H)


def _tc_layer1(s0, s1, cnt, x, wl_t, wr_t, b):
    def body(s0_r, s1_r, cnt_r, x_r, wl_r, wr_r, b_r, o_r, op_r):
        cn = jnp.maximum(cnt_r[...], 1.0)
        agg = jnp.concatenate([s0_r[...], s1_r[...]], axis=1) / cn
        acc = jnp.dot(agg, wl_r[...], preferred_element_type=jnp.float32)
        acc += jnp.dot(x_r[...], wr_r[...], preferred_element_type=jnp.float32)
        h = jnp.maximum(acc + b_r[...], 0.0)
        o_r[...] = h
        op_r[...] = _pack_bf16(h, _DW)

    grid = _N // _RB
    return pl.pallas_call(
        body,
        grid=(grid,),
        in_specs=[
            pl.BlockSpec((_RB, _DW), lambda i: (i, 0)),
            pl.BlockSpec((_RB, _DW), lambda i: (i, 0)),
            pl.BlockSpec((_RB, 1), lambda i: (i, 0)),
            pl.BlockSpec((_RB, _D), lambda i: (i, 0)),
            pl.BlockSpec((_D, _H), lambda i: (0, 0)),
            pl.BlockSpec((_D, _H), lambda i: (0, 0)),
            pl.BlockSpec((1, _H), lambda i: (0, 0)),
        ],
        out_specs=[
            pl.BlockSpec((_RB, _H), lambda i: (i, 0)),
            pl.BlockSpec((_RB, _H // 2), lambda i: (i, 0)),
        ],
        out_shape=[
            jax.ShapeDtypeStruct((_N, _H), jnp.float32),
            jax.ShapeDtypeStruct((_N, _H // 2), jnp.int32),
        ],
    )(s0, s1, cnt, x, wl_t, wr_t, b)


def _tc_layer2_ab(sq, cnt, h, wl2_t, wr_t, b2, w1a_t, w1b_t, b1m):
    def body(s0_r, s1_r, s2_r, s3_r, cnt_r, h_r, wl_r, wr_r, b2_r, w1a_r,
             w1b_r, b1_r, oa_r, ob_r):
        cn = jnp.maximum(cnt_r[...], 1.0)
        agg = jnp.concatenate(
            [s0_r[...], s1_r[...], s2_r[...], s3_r[...]], axis=1) / cn
        emb = jnp.dot(agg, wl_r[...], preferred_element_type=jnp.float32)
        emb += jnp.dot(h_r[...], wr_r[...], preferred_element_type=jnp.float32)
        emb += b2_r[...]
        a_full = jnp.dot(emb, w1a_r[...],
                         preferred_element_type=jnp.float32) + b1_r[...]
        b_full = jnp.dot(emb, w1b_r[...],
                         preferred_element_type=jnp.float32)
        oa_r[...] = _pack_bf16_pair(a_full)
        ob_r[...] = _pack_bf16_pair(b_full)

    grid = _N // _RB
    return pl.pallas_call(
        body,
        grid=(grid,),
        in_specs=[
            pl.BlockSpec((_RB, _DW), lambda i: (i, 0)),
            pl.BlockSpec((_RB, _DW), lambda i: (i, 0)),
            pl.BlockSpec((_RB, _DW), lambda i: (i, 0)),
            pl.BlockSpec((_RB, _DW), lambda i: (i, 0)),
            pl.BlockSpec((_RB, 1), lambda i: (i, 0)),
            pl.BlockSpec((_RB, _H), lambda i: (i, 0)),
            pl.BlockSpec((_H, _H), lambda i: (0, 0)),
            pl.BlockSpec((_H, _H), lambda i: (0, 0)),
            pl.BlockSpec((1, _H), lambda i: (0, 0)),
            pl.BlockSpec((_H, _H), lambda i: (0, 0)),
            pl.BlockSpec((_H, _H), lambda i: (0, 0)),
            pl.BlockSpec((1, _H), lambda i: (0, 0)),
        ],
        out_specs=[
            pl.BlockSpec((_RB, _H // 2), lambda i: (i, 0)),
            pl.BlockSpec((_RB, _H // 2), lambda i: (i, 0)),
        ],
        out_shape=[
            jax.ShapeDtypeStruct((_N, _H // 2), jnp.int32),
            jax.ShapeDtypeStruct((_N, _H // 2), jnp.int32),
        ],
    )(*sq, cnt, h, wl2_t, wr_t, b2, w1a_t, w1b_t, b1m)


def _tc_edge_c(ea, w1c_t):
    def body(ea_r, w_r, o_r):
        cv = jnp.dot(ea_r[...], w_r[...], preferred_element_type=jnp.float32)
        o_r[...] = _pack_bf16_pair(cv)

    eb = 4096
    return pl.pallas_call(
        body,
        grid=(_EPAD // eb,),
        in_specs=[
            pl.BlockSpec((eb, _DE), lambda i: (i, 0)),
            pl.BlockSpec((_DE, _H), lambda i: (0, 0)),
        ],
        out_specs=pl.BlockSpec((eb, _H // 2), lambda i: (i, 0)),
        out_shape=jax.ShapeDtypeStruct((_EPAD, _H // 2), jnp.int32),
    )(ea, w1c_t)


def kernel(x, edge_index, edge_attr, conv1_Wl, conv1_Wr, conv1_b,
           conv2_Wl, conv2_Wr, conv2_b, mlp_W1, mlp_b1, mlp_W2, mlp_b2):
    src = edge_index[0]
    dst = edge_index[1]
    pad = _EPAD - _E
    src_p = jnp.concatenate([src, jnp.zeros((pad,), jnp.int32)])
    dst_seg = jnp.concatenate([dst, jnp.full((pad,), _N, jnp.int32)])
    dst_p = jnp.concatenate([dst, jnp.zeros((pad,), jnp.int32)])
    srcr128 = src_p.reshape(_EPAD // 128, 128)
    dstr128 = dst_seg.reshape(_EPAD // 128, 128)
    dstr128e = dst_p.reshape(_EPAD // 128, 128)

    zrows = jnp.zeros((_NROWS, _DW), jnp.float32)
    zv = jnp.zeros((_NROWS,), jnp.float32)

    hw = _DW // 2
    s1h0, s1h1, cnt = _make_seg_sum(1, True, False)(
        x[:, :_DW], x[:, _DW:], srcr128, dstr128, zrows, zv)
    cnt2 = cnt.reshape(_NROWS, 1)
    h, hp = _tc_layer1(s1h0, s1h1, cnt2, x, conv1_Wl.T, conv1_Wr.T,
                       conv1_b.reshape(1, _H))

    sq = _make_seg_sum(2, False, True)(
        hp[:, :hw], hp[:, hw:2 * hw], hp[:, 2 * hw:3 * hw], hp[:, 3 * hw:],
        srcr128, dstr128, zrows, zv)
    w1a_t = mlp_W1[:, :_H].T
    w1b_t = mlp_W1[:, _H:2 * _H].T
    w1c_t = mlp_W1[:, 2 * _H:].T
    a_tab, b_tab = _tc_layer2_ab(
        sq, cnt2, h, conv2_Wl.T, conv2_Wr.T,
        conv2_b.reshape(1, _H), w1a_t, w1b_t, mlp_b1.reshape(1, _H))

    ea_p = jnp.concatenate(
        [edge_attr, jnp.zeros((pad, _DE), jnp.float32)], axis=0)
    c_tab = _tc_edge_c(ea_p, w1c_t)

    w2p = mlp_W2[0].reshape(2, _H // 32, _L).transpose(1, 0, 2).reshape(_H)
    aux = jnp.concatenate([w2p, mlp_b2, jnp.zeros((_L - 1,), jnp.float32)])
    logits = _make_edge_logits()(a_tab, b_tab, c_tab, srcr128, dstr128e, aux)
    return logits[:_E]
